# baseline (speedup 1.0000x reference)
import functools

import jax
import jax.numpy as jnp
from jax import lax
from jax.experimental import pallas as pl
from jax.experimental.pallas import tpu as pltpu
from jax.experimental.pallas import tpu_sc as plsc

_ROWS = 1000000
_COLS = 64
_B = 16384
_N = _B * _COLS
_NC = 2
_NS = 16
_NW = _NC * _NS
_PER_W = _N // _NW
_CHUNK = 128
_NCH = _PER_W // _CHUNK
_FIRE = 8
_L = 16

_JT = 7812
_R_TAIL = _JT * 128
_SLABW = _JT * 1024
_TAIL_OFF = 8 * _SLABW
_FLAT = _TAIL_OFF + _COLS * (_ROWS - _R_TAIL)
_TPW = _JT // 4


def _detile_body(dataT_hbm, lt_hbm, flat_hbm, sem):
    core = lax.axis_index("c")
    sid = lax.axis_index("s")
    i = core * 4 + lax.shift_right_logical(sid, 2)
    j0 = lax.bitwise_and(sid, 3) * _TPW
    base_row = pl.multiple_of((i * _JT + j0) * 8, 8)

    def _tile(t):
        return pltpu.make_async_copy(
            dataT_hbm.at[
                pl.ds(pl.multiple_of(i * 8, 8), 8),
                pl.ds((j0 + t) * 128, 128),
            ],
            flat_hbm.at[pl.ds(base_row + t * 8, 8)],
            sem,
        )

    @pl.loop(0, _TPW)
    def _go(t):
        _tile(t).start()

        @pl.when(t >= _FIRE)
        def _w():
            _tile(t - _FIRE).wait()

    @pl.loop(_TPW - _FIRE, _TPW)
    def _drain(t):
        _tile(t).wait()

    @pl.when((sid == 0) & (core == 0))
    def _lt():
        pltpu.sync_copy(
            lt_hbm, flat_hbm.at[pl.ds(_TAIL_OFF // 128, (_FLAT - _TAIL_OFF) // 128)]
        )


def _gather_body(data_hbm, idx_hbm, out_hbm, idx_v, val_v, sem):
    wid = lax.axis_index("s") * _NC + lax.axis_index("c")

    pltpu.sync_copy(idx_hbm.at[wid], idx_v)

    lane = lax.iota(jnp.int32, _L)
    cv_main = []
    cv_tail = []
    for q in range(4):
        c = lane + q * _L
        i = lax.shift_right_logical(c, 3)
        c2 = lax.bitwise_and(c, 7)
        cv_main.append(i * _SLABW + lax.shift_left(c2, 7))
        cv_tail.append(_TAIL_OFF - _R_TAIL + lax.shift_left(c, 6))

    @pl.loop(0, _NCH)
    def _flat(ch):
        for s in range(_CHUNK // _L):
            sl = (ch, pl.ds(s * _L, _L))
            r = idx_v[sl]
            q = s % 4
            a_main = (
                cv_main[q]
                + lax.shift_left(lax.shift_right_logical(r, 7), 10)
                + lax.bitwise_and(r, 127)
            )
            a_tail = cv_tail[q] + r
            idx_v[sl] = jnp.where(r < _R_TAIL, a_main, a_tail)

    @pl.loop(0, _NCH // _FIRE)
    def _gather(g):
        descs = []
        for b in range(_FIRE):
            ch = g * _FIRE + b
            descs.append(
                pltpu.async_copy(data_hbm.at[idx_v.at[ch]], val_v.at[ch], sem)
            )
        for d in descs:
            d.wait()

    @pl.loop(0, _NCH)
    def _double(ch):
        for s in range(_CHUNK // _L):
            sl = (ch, pl.ds(s * _L, _L))
            val_v[sl] = val_v[sl] * 2.0

    pltpu.sync_copy(val_v, out_hbm.at[wid])


@jax.jit
def _run(dataT, lt, idx3):
    mesh = plsc.VectorSubcoreMesh(core_axis_name="c", subcore_axis_name="s")
    detile = functools.partial(
        pl.kernel,
        out_type=jax.ShapeDtypeStruct((_FLAT // 128, 128), jnp.float32),
        mesh=mesh,
        scratch_types=[pltpu.SemaphoreType.DMA],
        compiler_params=pltpu.CompilerParams(use_tc_tiling_on_sc=True),
    )(_detile_body)
    flat = detile(dataT, lt).reshape(_FLAT)

    gather = functools.partial(
        pl.kernel,
        out_type=jax.ShapeDtypeStruct((_NW, _NCH, _CHUNK), jnp.float32),
        mesh=mesh,
        scratch_types=[
            pltpu.VMEM((_NCH, _CHUNK), jnp.int32),
            pltpu.VMEM((_NCH, _CHUNK), jnp.float32),
            pltpu.SemaphoreType.DMA,
        ],
    )(_gather_body)
    return gather(flat, idx3)


def kernel(data, indices):
    dataT = data.T
    lt = dataT[:, _R_TAIL:].reshape((_COLS * (_ROWS - _R_TAIL)) // 128, 128)
    idx3 = indices.astype(jnp.int32).reshape(_NW, _NCH, _CHUNK)
    out = _run(dataT, lt, idx3)
    return out.reshape(_B, _COLS)


# speedup vs baseline: 1.0971x; 1.0971x over previous
import functools

import jax
import jax.numpy as jnp
from jax import lax
from jax.experimental import pallas as pl
from jax.experimental.pallas import tpu as pltpu
from jax.experimental.pallas import tpu_sc as plsc

_ROWS = 1000000
_COLS = 64
_B = 16384
_N = _B * _COLS
_NC = 2
_NS = 16
_NW = _NC * _NS
_PER_W = _N // _NW
_CHUNK = 128
_NCH = _PER_W // _CHUNK
_FIRE = 8
_L = 16


def _body(data_hbm, idx_hbm, out_hbm, idx_v, val_v, sem):
    wid = lax.axis_index("s") * _NC + lax.axis_index("c")

    pltpu.sync_copy(idx_hbm.at[wid], idx_v)

    lane = lax.iota(jnp.int32, _L)

    @pl.loop(0, _NCH)
    def _flat(ch):
        for s in range(_CHUNK // _L):
            col = (s % 4) * _L
            sl = (ch, pl.ds(s * _L, _L))
            idx_v[sl] = idx_v[sl] + (lane + col) * _ROWS

    @pl.loop(0, _NCH // _FIRE)
    def _gather(g):
        descs = []
        for b in range(_FIRE):
            ch = g * _FIRE + b
            descs.append(
                pltpu.async_copy(data_hbm.at[idx_v.at[ch]], val_v.at[ch], sem)
            )
        for d in descs:
            d.wait()

    @pl.loop(0, _NCH)
    def _double(ch):
        for s in range(_CHUNK // _L):
            sl = (ch, pl.ds(s * _L, _L))
            val_v[sl] = val_v[sl] * 2.0

    pltpu.sync_copy(val_v, out_hbm.at[wid])


@jax.jit
def _run(data_flat, idx3):
    mesh = plsc.VectorSubcoreMesh(core_axis_name="c", subcore_axis_name="s")
    k = functools.partial(
        pl.kernel,
        out_type=jax.ShapeDtypeStruct((_NW, _NCH, _CHUNK), jnp.float32),
        mesh=mesh,
        scratch_types=[
            pltpu.VMEM((_NCH, _CHUNK), jnp.int32),
            pltpu.VMEM((_NCH, _CHUNK), jnp.float32),
            pltpu.SemaphoreType.DMA,
        ],
    )(_body)
    return k(data_flat, idx3)


def kernel(data, indices):
    flat2 = lax.optimization_barrier(data.T.reshape(_ROWS * _COLS // 128, 128))
    data_flat = flat2.reshape(_ROWS * _COLS)
    idx3 = indices.astype(jnp.int32).reshape(_NW, _NCH, _CHUNK)
    out = _run(data_flat, idx3)
    return out.reshape(_B, _COLS)


# speedup vs baseline: 11.1419x; 10.1557x over previous
import functools

import jax
import jax.numpy as jnp
from jax import lax
from jax.experimental import pallas as pl
from jax.experimental.pallas import tpu as pltpu
from jax.experimental.pallas import tpu_sc as plsc

_ROWS = 1000000
_COLS = 64
_B = 16384
_N = _B * _COLS
_NC = 2
_NS = 16
_NW = _NC * _NS
_PER_W = _N // _NW
_CHUNK = 128
_NCH = _PER_W // _CHUNK
_FIRE = 8
_L = 16


def _body(data_hbm, idx_hbm, out_hbm, idx_v, val_v, sem):
    wid = lax.axis_index("s") * _NC + lax.axis_index("c")

    pltpu.sync_copy(idx_hbm.at[wid], idx_v)

    lane = lax.iota(jnp.int32, _L)

    @pl.loop(0, _NCH)
    def _flat(ch):
        for s in range(_CHUNK // _L):
            col = (s % 4) * _L
            sl = (ch, pl.ds(s * _L, _L))
            idx_v[sl] = idx_v[sl] * _COLS + (lane + col)

    @pl.loop(0, _NCH // _FIRE)
    def _gather(g):
        descs = []
        for b in range(_FIRE):
            ch = g * _FIRE + b
            descs.append(
                pltpu.async_copy(
                    data_hbm.at[idx_v.at[ch]],
                    val_v.at[ch],
                    sem,
                )
            )
        for d in descs:
            d.wait()

    @pl.loop(0, _NCH)
    def _double(ch):
        for s in range(_CHUNK // _L):
            sl = (ch, pl.ds(s * _L, _L))
            val_v[sl] = val_v[sl] * 2.0

    pltpu.sync_copy(val_v, out_hbm.at[wid])


@jax.jit
def _run(data_flat, idx3):
    mesh = plsc.VectorSubcoreMesh(core_axis_name="c", subcore_axis_name="s")
    k = functools.partial(
        pl.kernel,
        out_type=jax.ShapeDtypeStruct((_NW, _NCH, _CHUNK), jnp.float32),
        mesh=mesh,
        scratch_types=[
            pltpu.VMEM((_NCH, _CHUNK), jnp.int32),
            pltpu.VMEM((_NCH, _CHUNK), jnp.float32),
            pltpu.SemaphoreType.DMA,
        ],
    )(_body)
    return k(data_flat, idx3)


def kernel(data, indices):
    data_flat = data.reshape(_ROWS * _COLS)
    idx3 = indices.astype(jnp.int32).reshape(_NW, _NCH, _CHUNK)
    out = _run(data_flat, idx3)
    return out.reshape(_B, _COLS)


# speedup vs baseline: 12.2888x; 1.1029x over previous
import functools

import jax
import jax.numpy as jnp
from jax import lax
from jax.experimental import pallas as pl
from jax.experimental.pallas import tpu as pltpu
from jax.experimental.pallas import tpu_sc as plsc

_ROWS = 1000000
_COLS = 64
_B = 16384
_N = _B * _COLS
_NC = 2
_NS = 16
_NW = _NC * _NS
_PER_W = _N // _NW
_CHUNK = 128
_NCH = _PER_W // _CHUNK
_FIRE = 8
_L = 16


def _body(data_hbm, idx_hbm, out_hbm, idx_v, val_v, sem):
    wid = lax.axis_index("s") * _NC + lax.axis_index("c")

    pltpu.sync_copy(idx_hbm.at[wid], idx_v)

    lane = lax.iota(jnp.int32, _L)

    @pl.loop(0, _NCH)
    def _flat(ch):
        for s in range(_CHUNK // _L):
            col = (s % 4) * _L
            sl = (ch, pl.ds(s * _L, _L))
            idx_v[sl] = lax.shift_left(idx_v[sl], 7) + (lane + col)

    @pl.loop(0, _NCH // _FIRE)
    def _gather(g):
        descs = []
        for b in range(_FIRE):
            ch = g * _FIRE + b
            descs.append(
                pltpu.async_copy(data_hbm.at[idx_v.at[ch]], val_v.at[ch], sem)
            )
        for d in descs:
            d.wait()

    @pl.loop(0, _NCH)
    def _double(ch):
        for s in range(_CHUNK // _L):
            sl = (ch, pl.ds(s * _L, _L))
            val_v[sl] = val_v[sl] * 2.0

    pltpu.sync_copy(val_v, out_hbm.at[wid])


@jax.jit
def _run(data_flat, idx3):
    mesh = plsc.VectorSubcoreMesh(core_axis_name="c", subcore_axis_name="s")
    k = functools.partial(
        pl.kernel,
        out_type=jax.ShapeDtypeStruct((_NW, _NCH, _CHUNK), jnp.float32),
        mesh=mesh,
        scratch_types=[
            pltpu.VMEM((_NCH, _CHUNK), jnp.int32),
            pltpu.VMEM((_NCH, _CHUNK), jnp.float32),
            pltpu.SemaphoreType.DMA,
        ],
    )(_body)
    return k(data_flat, idx3)


def kernel(data, indices):
    data_pad = jnp.pad(data, ((0, 0), (0, 64)))
    data_flat = data_pad.reshape(_ROWS * 128)
    idx3 = indices.astype(jnp.int32).reshape(_NW, _NCH, _CHUNK)
    out = _run(data_flat, idx3)
    return out.reshape(_B, _COLS)


# speedup vs baseline: 12.3274x; 1.0031x over previous
import functools

import jax
import jax.numpy as jnp
from jax import lax
from jax.experimental import pallas as pl
from jax.experimental.pallas import tpu as pltpu
from jax.experimental.pallas import tpu_sc as plsc

_ROWS = 1000000
_COLS = 64
_B = 16384
_N = _B * _COLS
_NC = 2
_NS = 16
_NW = _NC * _NS
_PER_W = _N // _NW
_CHUNK = 128
_NCH = _PER_W // _CHUNK
_FIRE = 8
_L = 16


def _body(data_hbm, idx_hbm, out_hbm, idx_v, val_v, sem):
    wid = lax.axis_index("s") * _NC + lax.axis_index("c")

    pltpu.sync_copy(idx_hbm.at[wid], idx_v)

    lane = lax.iota(jnp.int32, _L)

    def _addr(ch):
        for s in range(_CHUNK // _L):
            col = (s % 4) * _L
            sl = (ch, pl.ds(s * _L, _L))
            idx_v[sl] = lax.shift_left(idx_v[sl], 7) + (lane + col)

    def _dbl(ch):
        for s in range(_CHUNK // _L):
            sl = (ch, pl.ds(s * _L, _L))
            val_v[sl] = val_v[sl] * 2.0

    @pl.loop(0, _FIRE)
    def _pro(ch):
        _addr(ch)

    ngrp = _NCH // _FIRE

    @pl.loop(0, ngrp)
    def _gather(g):
        descs = []
        for b in range(_FIRE):
            ch = g * _FIRE + b
            descs.append(
                pltpu.async_copy(data_hbm.at[idx_v.at[ch]], val_v.at[ch], sem)
            )

        @pl.when(g + 1 < ngrp)
        def _a():
            @pl.loop((g + 1) * _FIRE, (g + 2) * _FIRE)
            def _aa(ch):
                _addr(ch)

        @pl.when(g >= 1)
        def _d():
            @pl.loop((g - 1) * _FIRE, g * _FIRE)
            def _dd(ch):
                _dbl(ch)

        for d in descs:
            d.wait()

    @pl.loop((ngrp - 1) * _FIRE, ngrp * _FIRE)
    def _epi(ch):
        _dbl(ch)

    pltpu.sync_copy(val_v, out_hbm.at[wid])


@jax.jit
def _run(data_flat, idx3):
    mesh = plsc.VectorSubcoreMesh(core_axis_name="c", subcore_axis_name="s")
    k = functools.partial(
        pl.kernel,
        out_type=jax.ShapeDtypeStruct((_NW, _NCH, _CHUNK), jnp.float32),
        mesh=mesh,
        scratch_types=[
            pltpu.VMEM((_NCH, _CHUNK), jnp.int32),
            pltpu.VMEM((_NCH, _CHUNK), jnp.float32),
            pltpu.SemaphoreType.DMA,
        ],
    )(_body)
    return k(data_flat, idx3)


def kernel(data, indices):
    data_pad = jnp.pad(data, ((0, 0), (0, 64)))
    data_flat = data_pad.reshape(_ROWS * 128)
    idx3 = indices.astype(jnp.int32).reshape(_NW, _NCH, _CHUNK)
    out = _run(data_flat, idx3)
    return out.reshape(_B, _COLS)
